# baseline (speedup 1.0000x reference)
import functools

import numpy as np
import jax
import jax.numpy as jnp
from jax import lax
from jax.experimental import pallas as pl
from jax.experimental.pallas import tpu as pltpu
from jax.experimental.pallas import tpu_sc as plsc

_H = 128
_NC = 2
_NS = 16
_CH = 96
_NSC = 20


def _deinterleave_perm(h):
    r = np.arange(h)
    g = 32 * (r // 32)
    m = r % 32
    return np.where(m % 2 == 0, g + m // 2, g + 16 + m // 2)


def _dense_f32_body(x_ref, w_ref, b_ref, o_ref):
    o_ref[0] = (
        jnp.dot(x_ref[...], w_ref[...], preferred_element_type=jnp.float32)
        + b_ref[0]
    )


def _dense_bf16_body(x_ref, w_ref, b_ref, o_ref):
    o_ref[0] = (
        jnp.dot(x_ref[...], w_ref[...], preferred_element_type=jnp.float32)
        + b_ref[0]
    ).astype(jnp.bfloat16)


def _dense(x, w_cat, b_cat, nblk, body, out_dtype):
    n, h = x.shape
    g = w_cat.shape[1] // h
    ni = n // nblk
    return pl.pallas_call(
        body,
        grid=(ni, g),
        in_specs=[
            pl.BlockSpec((nblk, h), lambda i, t: (i, 0)),
            pl.BlockSpec((h, h), lambda i, t: (0, t)),
            pl.BlockSpec((1, 1, h), lambda i, t: (t, 0, 0)),
        ],
        out_specs=pl.BlockSpec((1, nblk, h), lambda i, t: (t, i, 0)),
        out_shape=jax.ShapeDtypeStruct((g, n, h), out_dtype),
    )(x, w_cat, b_cat)


def _gru_body(part_ref, xr_ref, xz_ref, xn_ref, whr_ref, whz_ref, whn_ref,
              bhn_ref, o_ref):
    p = part_ref[0] + part_ref[1]
    r = jax.nn.sigmoid(
        xr_ref[0] + jnp.dot(p, whr_ref[...], preferred_element_type=jnp.float32))
    z = jax.nn.sigmoid(
        xz_ref[0] + jnp.dot(p, whz_ref[...], preferred_element_type=jnp.float32))
    nn = jnp.tanh(
        xn_ref[0]
        + r * (jnp.dot(p, whn_ref[...], preferred_element_type=jnp.float32)
               + bhn_ref[...]))
    o_ref[...] = (1.0 - z) * nn + z * p


def _gru(part, x3, whr, whz, whn, bhn, nblk):
    n = x3.shape[1]
    h = x3.shape[2]
    ni = n // nblk
    return pl.pallas_call(
        _gru_body,
        grid=(ni,),
        in_specs=[
            pl.BlockSpec((2, nblk, h), lambda i: (0, i, 0)),
            pl.BlockSpec((1, nblk, h), lambda i: (0, i, 0)),
            pl.BlockSpec((1, nblk, h), lambda i: (1, i, 0)),
            pl.BlockSpec((1, nblk, h), lambda i: (2, i, 0)),
            pl.BlockSpec((h, h), lambda i: (0, 0)),
            pl.BlockSpec((h, h), lambda i: (0, 0)),
            pl.BlockSpec((h, h), lambda i: (0, 0)),
            pl.BlockSpec((1, h), lambda i: (0, 0)),
        ],
        out_specs=pl.BlockSpec((nblk, h), lambda i: (i, 0)),
        out_shape=jax.ShapeDtypeStruct((n, h), jnp.float32),
    )(part, x3, x3, x3, whr, whz, whn, bhn)


def _sc_segment_sum(ytab, src, dst, typ, zrows, n_nodes, npad, rpt):
    e = src.shape[0]
    hw = ytab.shape[1]
    h = hw * 2
    nw = _NC * _NS
    epw = e // nw
    se = _NSC * _CH
    nsup = epw // se
    rem = epw - nsup * se

    mesh = plsc.VectorSubcoreMesh(core_axis_name="c", subcore_axis_name="s")

    @functools.partial(
        pl.kernel,
        mesh=mesh,
        compiler_params=pltpu.CompilerParams(use_tc_tiling_on_sc=False),
        out_type=jax.ShapeDtypeStruct((_NC, npad, h), jnp.float32),
        scratch_types=[
            pltpu.VMEM((se,), jnp.int32),
            pltpu.VMEM((se,), jnp.int32),
            pltpu.VMEM((se,), jnp.int32),
            pltpu.VMEM((_NSC, _CH), jnp.int32),
            pltpu.VMEM((_NSC, _CH), jnp.int32),
            pltpu.VMEM((_CH, hw), jnp.int32),
            pltpu.VMEM((_CH, hw), jnp.int32),
            pltpu.VMEM((_CH, h), jnp.float32),
            pltpu.VMEM((_CH, h), jnp.float32),
            pltpu.VMEM_SHARED((npad, h), jnp.float32),
            pltpu.SemaphoreType.DMA,
            pltpu.SemaphoreType.DMA,
            pltpu.SemaphoreType.DMA,
            pltpu.SemaphoreType.DMA,
        ],
    )
    def sck(ytab_h, src_h, dst_h, typ_h, z_h, out_h,
            sflat, tflat, dflat, gi2d, dj2d, bf0, bf1, f0, f1, acc,
            gsem0, gsem1, ssem0, ssem1):
        cid = lax.axis_index("c")
        sid = lax.axis_index("s")
        wid = cid * _NS + sid
        base = wid * epw

        pltpu.sync_copy(z_h, acc.at[pl.ds(sid * rpt, rpt)])
        plsc.subcore_barrier()

        mask_hi = jnp.full((16,), -65536, jnp.int32)

        def widen(bfbuf, fbuf):
            def rbody(rr, carry):
                for gg in range(h // 32):
                    v = bfbuf[rr, pl.ds(16 * gg, 16)]
                    fbuf[rr, pl.ds(32 * gg, 16)] = lax.bitcast_convert_type(
                        v << 16, jnp.float32)
                    fbuf[rr, pl.ds(32 * gg + 16, 16)] = lax.bitcast_convert_type(
                        v & mask_hi, jnp.float32)
                return carry
            lax.fori_loop(0, _CH, rbody, 0)

        def superchunk(u, valid, nsc):
            pltpu.sync_copy(src_h.at[pl.ds(base + u * se, valid)],
                            sflat.at[pl.ds(0, valid)])
            pltpu.sync_copy(typ_h.at[pl.ds(base + u * se, valid)],
                            tflat.at[pl.ds(0, valid)])
            pltpu.sync_copy(dst_h.at[pl.ds(base + u * se, valid)],
                            dflat.at[pl.ds(0, valid)])

            def cbody(j, carry):
                for k in range(_CH // 16):
                    off = j * _CH + k * 16
                    s = sflat[pl.ds(off, 16)]
                    t = tflat[pl.ds(off, 16)]
                    gi2d[j, pl.ds(k * 16, 16)] = t * n_nodes + s
                    dj2d[j, pl.ds(k * 16, 16)] = dflat[pl.ds(off, 16)]
                return carry
            lax.fori_loop(0, nsc, cbody, 0)

            zero16 = jnp.zeros((16,), jnp.int32)
            junk16 = jnp.full((16,), n_nodes, jnp.int32)
            for m in range(valid // 16, nsc * _CH // 16):
                j, k = m // (_CH // 16), m % (_CH // 16)
                gi2d[j, pl.ds(k * 16, 16)] = zero16
                dj2d[j, pl.ds(k * 16, 16)] = junk16

            def pbody(i, carry):
                a = 2 * i
                b = a + 1
                ga = pltpu.async_copy(ytab_h.at[gi2d.at[a]], bf0, gsem0)
                gb = pltpu.async_copy(ytab_h.at[gi2d.at[b]], bf1, gsem1)
                ga.wait()
                widen(bf0, f0)
                sa = pltpu.async_copy(f0, acc.at[dj2d.at[a]], ssem0, add=True)
                gb.wait()
                widen(bf1, f1)
                sb = pltpu.async_copy(f1, acc.at[dj2d.at[b]], ssem1, add=True)
                sa.wait()
                sb.wait()
                return carry
            lax.fori_loop(0, nsc // 2, pbody, 0)

        for u in range(nsup):
            superchunk(u, se, _NSC)
        if rem:
            nsc_last = -(-rem // _CH)
            nsc_last = nsc_last + (nsc_last % 2)
            superchunk(nsup, rem, nsc_last)

        plsc.subcore_barrier()
        pltpu.sync_copy(acc.at[pl.ds(sid * rpt, rpt)],
                        out_h.at[cid, pl.ds(sid * rpt, rpt)])

    return sck(ytab, src, dst, typ, zrows)


def kernel(node_embeddings, source_indices, dest_indices, edge_types,
           num_edges, W_e, b_e, W_ir, b_ir, W_hr, W_iz, b_iz, W_hz, W_in,
           b_in, W_hn, b_hn):
    n, h = node_embeddings.shape
    t = W_e.shape[1] // h
    del num_edges

    rpt = ((n + _NS - 1) // _NS + 7) // 8 * 8
    npad = rpt * _NS

    perm = _deinterleave_perm(h)
    perm_full = jnp.asarray(
        np.concatenate([tt * h + perm for tt in range(t)]))
    w_e_perm = jnp.take(W_e, perm_full, axis=1)
    b_e_perm = jnp.take(b_e, perm_full).reshape(t, 1, h)

    y6 = _dense(node_embeddings, w_e_perm, b_e_perm, 1000,
                _dense_bf16_body, jnp.bfloat16)
    ytab = lax.bitcast_convert_type(
        y6.reshape(t * n, h // 2, 2), jnp.int32)

    w_i3 = jnp.concatenate([W_ir, W_iz, W_in], axis=1)
    b_i3 = jnp.concatenate([b_ir, b_iz, b_in]).reshape(3, 1, h)
    x3 = _dense(node_embeddings, w_i3, b_i3, 1000,
                _dense_f32_body, jnp.float32)

    zrows = jnp.zeros((rpt, h), jnp.float32)
    part = _sc_segment_sum(ytab, source_indices, dest_indices, edge_types,
                           zrows, n, npad, rpt)

    return _gru(part, x3, W_hr, W_hz, W_hn, b_hn.reshape(1, h), nblk=1000)


# speedup vs baseline: 1.2711x; 1.2711x over previous
import functools

import jax
import jax.numpy as jnp
from jax import lax
from jax.experimental import pallas as pl
from jax.experimental.pallas import tpu as pltpu
from jax.experimental.pallas import tpu_sc as plsc

_H = 128
_NC = 2
_NS = 16
_CH = 128


def _dense_body(x_ref, w_ref, b_ref, o_ref):
    o_ref[0] = (
        jnp.dot(x_ref[...], w_ref[...], preferred_element_type=jnp.float32)
        + b_ref[0]
    )


def _edge_transform(x, w_cat, b_cat, nblk):
    n, h = x.shape
    g = w_cat.shape[1] // h
    ni = n // nblk
    return pl.pallas_call(
        _dense_body,
        grid=(ni, g),
        in_specs=[
            pl.BlockSpec((nblk, h), lambda i, t: (i, 0)),
            pl.BlockSpec((h, h), lambda i, t: (0, t)),
            pl.BlockSpec((1, 1, h), lambda i, t: (t, 0, 0)),
        ],
        out_specs=pl.BlockSpec((1, nblk, h), lambda i, t: (t, i, 0)),
        out_shape=jax.ShapeDtypeStruct((g, n, h), jnp.float32),
    )(x, w_cat, b_cat)


def _gru_body(part_ref, xr_ref, xz_ref, xn_ref, whr_ref, whz_ref, whn_ref,
              bhn_ref, o_ref):
    p = part_ref[0] + part_ref[1]
    r = jax.nn.sigmoid(
        xr_ref[0] + jnp.dot(p, whr_ref[...], preferred_element_type=jnp.float32))
    z = jax.nn.sigmoid(
        xz_ref[0] + jnp.dot(p, whz_ref[...], preferred_element_type=jnp.float32))
    nn = jnp.tanh(
        xn_ref[0]
        + r * (jnp.dot(p, whn_ref[...], preferred_element_type=jnp.float32)
               + bhn_ref[...]))
    o_ref[...] = (1.0 - z) * nn + z * p


def _gru(part, y, whr, whz, whn, bhn, nblk):
    npad = part.shape[1]
    n = y.shape[1]
    h = y.shape[2]
    ni = n // nblk
    return pl.pallas_call(
        _gru_body,
        grid=(ni,),
        in_specs=[
            pl.BlockSpec((2, nblk, h), lambda i: (0, i, 0)),
            pl.BlockSpec((1, nblk, h), lambda i: (6, i, 0)),
            pl.BlockSpec((1, nblk, h), lambda i: (7, i, 0)),
            pl.BlockSpec((1, nblk, h), lambda i: (8, i, 0)),
            pl.BlockSpec((h, h), lambda i: (0, 0)),
            pl.BlockSpec((h, h), lambda i: (0, 0)),
            pl.BlockSpec((h, h), lambda i: (0, 0)),
            pl.BlockSpec((1, h), lambda i: (0, 0)),
        ],
        out_specs=pl.BlockSpec((nblk, h), lambda i: (i, 0)),
        out_shape=jax.ShapeDtypeStruct((n, h), jnp.float32),
    )(part, y, y, y, whr, whz, whn, bhn)


def _sc_segment_sum(ytab, src, dst, typ, zrows, n_nodes, npad, rpt):
    e = src.shape[0]
    h = ytab.shape[1]
    nw = _NC * _NS
    epw = e // nw
    nsc = 20
    se = nsc * _CH
    nsup = -(-epw // se)

    mesh = plsc.VectorSubcoreMesh(core_axis_name="c", subcore_axis_name="s")

    @functools.partial(
        pl.kernel,
        mesh=mesh,
        out_type=jax.ShapeDtypeStruct((_NC, npad, h), jnp.float32),
        scratch_types=[
            pltpu.VMEM((se,), jnp.int32),
            pltpu.VMEM((se,), jnp.int32),
            pltpu.VMEM((se,), jnp.int32),
            pltpu.VMEM((nsc, _CH), jnp.int32),
            pltpu.VMEM((nsc, _CH), jnp.int32),
            pltpu.VMEM((_CH, h), jnp.float32),
            pltpu.VMEM((_CH, h), jnp.float32),
            pltpu.VMEM_SHARED((npad, h), jnp.float32),
            pltpu.SemaphoreType.DMA,
            pltpu.SemaphoreType.DMA,
            pltpu.SemaphoreType.DMA,
            pltpu.SemaphoreType.DMA,
        ],
    )
    def sck(ytab_h, src_h, dst_h, typ_h, z_h, out_h,
            sflat, tflat, dflat, gi2d, dj2d, rows0, rows1, acc,
            sem0, sem1, sem2, sem3):
        cid = lax.axis_index("c")
        sid = lax.axis_index("s")
        wid = cid * _NS + sid
        base = wid * epw

        pltpu.sync_copy(z_h, acc.at[pl.ds(sid * rpt, rpt)])
        plsc.subcore_barrier()

        zero16 = jnp.zeros((16,), jnp.int32)
        junk16 = jnp.full((16,), n_nodes, jnp.int32)

        for u in range(nsup):
            valid = min(se, epw - u * se)
            pltpu.sync_copy(src_h.at[pl.ds(base + u * se, valid)],
                            sflat.at[pl.ds(0, valid)])
            pltpu.sync_copy(typ_h.at[pl.ds(base + u * se, valid)],
                            tflat.at[pl.ds(0, valid)])
            pltpu.sync_copy(dst_h.at[pl.ds(base + u * se, valid)],
                            dflat.at[pl.ds(0, valid)])

            def cbody(j, carry):
                for k in range(_CH // 16):
                    off = j * _CH + k * 16
                    s = sflat[pl.ds(off, 16)]
                    t = tflat[pl.ds(off, 16)]
                    gi2d[j, pl.ds(k * 16, 16)] = t * n_nodes + s
                    dj2d[j, pl.ds(k * 16, 16)] = dflat[pl.ds(off, 16)]
                return carry
            lax.fori_loop(0, nsc, cbody, 0)

            for m in range(valid // 16, se // 16):
                j, k = m // (_CH // 16), m % (_CH // 16)
                gi2d[j, pl.ds(k * 16, 16)] = zero16
                dj2d[j, pl.ds(k * 16, 16)] = junk16

            def pbody(i, carry):
                c0 = i * 4
                ga = pltpu.async_copy(ytab_h.at[gi2d.at[c0]], rows0, sem0)
                gb = pltpu.async_copy(ytab_h.at[gi2d.at[c0 + 1]], rows1, sem1)
                ga.wait()
                sa = pltpu.async_copy(rows0, acc.at[dj2d.at[c0]], sem2,
                                      add=True)
                gb.wait()
                sb = pltpu.async_copy(rows1, acc.at[dj2d.at[c0 + 1]], sem3,
                                      add=True)
                sa.wait()
                gc = pltpu.async_copy(ytab_h.at[gi2d.at[c0 + 2]], rows0, sem0)
                sb.wait()
                gd = pltpu.async_copy(ytab_h.at[gi2d.at[c0 + 3]], rows1, sem1)
                gc.wait()
                se_ = pltpu.async_copy(rows0, acc.at[dj2d.at[c0 + 2]], sem2,
                                       add=True)
                gd.wait()
                sf = pltpu.async_copy(rows1, acc.at[dj2d.at[c0 + 3]], sem3,
                                      add=True)
                se_.wait()
                sf.wait()
                return carry
            lax.fori_loop(0, nsc // 4, pbody, 0)

        plsc.subcore_barrier()
        pltpu.sync_copy(acc.at[pl.ds(sid * rpt, rpt)],
                        out_h.at[cid, pl.ds(sid * rpt, rpt)])

    return sck(ytab, src, dst, typ, zrows)


def kernel(node_embeddings, source_indices, dest_indices, edge_types,
           num_edges, W_e, b_e, W_ir, b_ir, W_hr, W_iz, b_iz, W_hz, W_in,
           b_in, W_hn, b_hn):
    n, h = node_embeddings.shape
    del num_edges

    rpt = ((n + _NS - 1) // _NS + 7) // 8 * 8
    npad = rpt * _NS

    w_cat = jnp.concatenate([W_e, W_ir, W_iz, W_in], axis=1)
    b_cat = jnp.concatenate([b_e, b_ir, b_iz, b_in]).reshape(-1, 1, h)

    y = _edge_transform(node_embeddings, w_cat, b_cat, nblk=1000)
    ytab = y.reshape(-1, h)

    zrows = jnp.zeros((rpt, h), jnp.float32)
    part = _sc_segment_sum(ytab, source_indices, dest_indices, edge_types,
                           zrows, n, npad, rpt)

    return _gru(part, y, W_hr, W_hz, W_hn, b_hn.reshape(1, h), nblk=1000)


# speedup vs baseline: 1.3330x; 1.0487x over previous
import functools

import jax
import jax.numpy as jnp
from jax import lax
from jax.experimental import pallas as pl
from jax.experimental.pallas import tpu as pltpu
from jax.experimental.pallas import tpu_sc as plsc

_H = 128
_NC = 2
_NS = 16
_CH = 64


def _dense_body(x_ref, w_ref, b_ref, o_ref):
    o_ref[0] = (
        jnp.dot(x_ref[...], w_ref[...], preferred_element_type=jnp.float32)
        + b_ref[0]
    )


def _edge_transform(x, w_cat, b_cat, nblk):
    n, h = x.shape
    g = w_cat.shape[1] // h
    ni = n // nblk
    return pl.pallas_call(
        _dense_body,
        grid=(ni, g),
        in_specs=[
            pl.BlockSpec((nblk, h), lambda i, t: (i, 0)),
            pl.BlockSpec((h, h), lambda i, t: (0, t)),
            pl.BlockSpec((1, 1, h), lambda i, t: (t, 0, 0)),
        ],
        out_specs=pl.BlockSpec((1, nblk, h), lambda i, t: (t, i, 0)),
        out_shape=jax.ShapeDtypeStruct((g, n, h), jnp.float32),
    )(x, w_cat, b_cat)


def _gru_body(part_ref, xr_ref, xz_ref, xn_ref, whr_ref, whz_ref, whn_ref,
              bhn_ref, o_ref):
    p = part_ref[0] + part_ref[1]
    r = jax.nn.sigmoid(
        xr_ref[0] + jnp.dot(p, whr_ref[...], preferred_element_type=jnp.float32))
    z = jax.nn.sigmoid(
        xz_ref[0] + jnp.dot(p, whz_ref[...], preferred_element_type=jnp.float32))
    nn = jnp.tanh(
        xn_ref[0]
        + r * (jnp.dot(p, whn_ref[...], preferred_element_type=jnp.float32)
               + bhn_ref[...]))
    o_ref[...] = (1.0 - z) * nn + z * p


def _gru(part, y, whr, whz, whn, bhn, nblk):
    npad = part.shape[1]
    n = y.shape[1]
    h = y.shape[2]
    ni = n // nblk
    return pl.pallas_call(
        _gru_body,
        grid=(ni,),
        in_specs=[
            pl.BlockSpec((2, nblk, h), lambda i: (0, i, 0)),
            pl.BlockSpec((1, nblk, h), lambda i: (6, i, 0)),
            pl.BlockSpec((1, nblk, h), lambda i: (7, i, 0)),
            pl.BlockSpec((1, nblk, h), lambda i: (8, i, 0)),
            pl.BlockSpec((h, h), lambda i: (0, 0)),
            pl.BlockSpec((h, h), lambda i: (0, 0)),
            pl.BlockSpec((h, h), lambda i: (0, 0)),
            pl.BlockSpec((1, h), lambda i: (0, 0)),
        ],
        out_specs=pl.BlockSpec((nblk, h), lambda i: (i, 0)),
        out_shape=jax.ShapeDtypeStruct((n, h), jnp.float32),
    )(part, y, y, y, whr, whz, whn, bhn)


def _sc_segment_sum(ytab, src, dst, typ, zrows, n_nodes, npad, rpt):
    e = src.shape[0]
    h = ytab.shape[1]
    nw = _NC * _NS
    epw = e // nw
    nsc = 32
    se = nsc * _CH
    nsup = -(-epw // se)

    mesh = plsc.VectorSubcoreMesh(core_axis_name="c", subcore_axis_name="s")

    @functools.partial(
        pl.kernel,
        mesh=mesh,
        out_type=jax.ShapeDtypeStruct((_NC, npad, h), jnp.float32),
        scratch_types=[
            pltpu.VMEM((se,), jnp.int32),
            pltpu.VMEM((se,), jnp.int32),
            pltpu.VMEM((se,), jnp.int32),
            pltpu.VMEM((nsc, _CH), jnp.int32),
            pltpu.VMEM((nsc, _CH), jnp.int32),
            pltpu.VMEM((_CH, h), jnp.float32),
            pltpu.VMEM((_CH, h), jnp.float32),
            pltpu.VMEM((_CH, h), jnp.float32),
            pltpu.VMEM((_CH, h), jnp.float32),
            pltpu.VMEM_SHARED((npad, h), jnp.float32),
            pltpu.SemaphoreType.DMA,
            pltpu.SemaphoreType.DMA,
            pltpu.SemaphoreType.DMA,
            pltpu.SemaphoreType.DMA,
            pltpu.SemaphoreType.DMA,
            pltpu.SemaphoreType.DMA,
            pltpu.SemaphoreType.DMA,
            pltpu.SemaphoreType.DMA,
        ],
    )
    def sck(ytab_h, src_h, dst_h, typ_h, z_h, out_h,
            sflat, tflat, dflat, gi2d, dj2d, rows0, rows1, rows2, rows3, acc,
            gsem0, gsem1, gsem2, gsem3, ssem0, ssem1, ssem2, ssem3):
        rows = (rows0, rows1, rows2, rows3)
        gsem = (gsem0, gsem1, gsem2, gsem3)
        ssem = (ssem0, ssem1, ssem2, ssem3)
        cid = lax.axis_index("c")
        sid = lax.axis_index("s")
        wid = cid * _NS + sid
        base = wid * epw

        pltpu.sync_copy(z_h, acc.at[pl.ds(sid * rpt, rpt)])
        plsc.subcore_barrier()

        zero16 = jnp.zeros((16,), jnp.int32)
        junk16 = jnp.full((16,), n_nodes, jnp.int32)

        for u in range(nsup):
            valid = min(se, epw - u * se)
            pltpu.sync_copy(src_h.at[pl.ds(base + u * se, valid)],
                            sflat.at[pl.ds(0, valid)])
            pltpu.sync_copy(typ_h.at[pl.ds(base + u * se, valid)],
                            tflat.at[pl.ds(0, valid)])
            pltpu.sync_copy(dst_h.at[pl.ds(base + u * se, valid)],
                            dflat.at[pl.ds(0, valid)])

            def cbody(j, carry):
                for k in range(_CH // 16):
                    off = j * _CH + k * 16
                    s = sflat[pl.ds(off, 16)]
                    t = tflat[pl.ds(off, 16)]
                    gi2d[j, pl.ds(k * 16, 16)] = t * n_nodes + s
                    dj2d[j, pl.ds(k * 16, 16)] = dflat[pl.ds(off, 16)]
                return carry
            lax.fori_loop(0, nsc, cbody, 0)

            for m in range(valid // 16, se // 16):
                j, k = m // (_CH // 16), m % (_CH // 16)
                gi2d[j, pl.ds(k * 16, 16)] = zero16
                dj2d[j, pl.ds(k * 16, 16)] = junk16

            for b in range(4):
                pltpu.async_copy(ytab_h.at[gi2d.at[b]], rows[b], gsem[b])

            def rbody(i, carry):
                c0 = 4 * i
                for b in range(4):
                    pltpu.make_async_copy(
                        ytab_h.at[gi2d.at[b]], rows[b], gsem[b]).wait()
                    pltpu.async_copy(rows[b], acc.at[dj2d.at[c0 + b]],
                                     ssem[b], add=True)
                for b in range(4):
                    pltpu.make_async_copy(
                        rows[b], acc.at[dj2d.at[b]], ssem[b]).wait()
                    pltpu.async_copy(ytab_h.at[gi2d.at[c0 + 4 + b]],
                                     rows[b], gsem[b])
                return carry
            lax.fori_loop(0, nsc // 4 - 1, rbody, 0)

            c0 = nsc - 4
            for b in range(4):
                pltpu.make_async_copy(
                    ytab_h.at[gi2d.at[b]], rows[b], gsem[b]).wait()
                pltpu.async_copy(rows[b], acc.at[dj2d.at[c0 + b]],
                                 ssem[b], add=True)
            for b in range(4):
                pltpu.make_async_copy(
                    rows[b], acc.at[dj2d.at[b]], ssem[b]).wait()

        plsc.subcore_barrier()
        pltpu.sync_copy(acc.at[pl.ds(sid * rpt, rpt)],
                        out_h.at[cid, pl.ds(sid * rpt, rpt)])

    return sck(ytab, src, dst, typ, zrows)


def kernel(node_embeddings, source_indices, dest_indices, edge_types,
           num_edges, W_e, b_e, W_ir, b_ir, W_hr, W_iz, b_iz, W_hz, W_in,
           b_in, W_hn, b_hn):
    n, h = node_embeddings.shape
    del num_edges

    rpt = ((n + _NS - 1) // _NS + 7) // 8 * 8
    npad = rpt * _NS

    w_cat = jnp.concatenate([W_e, W_ir, W_iz, W_in], axis=1)
    b_cat = jnp.concatenate([b_e, b_ir, b_iz, b_in]).reshape(-1, 1, h)

    y = _edge_transform(node_embeddings, w_cat, b_cat, nblk=1000)
    ytab = y.reshape(-1, h)

    zrows = jnp.zeros((rpt, h), jnp.float32)
    part = _sc_segment_sum(ytab, source_indices, dest_indices, edge_types,
                           zrows, n, npad, rpt)

    return _gru(part, y, W_hr, W_hz, W_hn, b_hn.reshape(1, h), nblk=1000)
